# Initial kernel scaffold; baseline (speedup 1.0000x reference)
import functools

import jax
import jax.numpy as jnp
from jax import lax
from jax.experimental import pallas as pl
from jax.experimental.pallas import tpu as pltpu
from jax.experimental.pallas import tpu_sc as plsc

N = 10000
NPAD = 10016
NACC = 10240
NT = 32
NCH = 80
CH = 128
EPT = NCH * CH
EPAD = NT * EPT
JUNK = N


def _sc_mesh():
    return plsc.VectorSubcoreMesh(
        core_axis_name="c", subcore_axis_name="s", num_cores=2, num_subcores=16
    )


def _deg_body(dst_hbm, rowid_hbm, zeros_hbm, out_hbm, dst_v, hist_v, rows_v, sdeg):
    cid = lax.axis_index("c")
    sid = lax.axis_index("s")
    wid = cid * 16 + sid
    pltpu.sync_copy(dst_hbm.at[wid], dst_v)
    pltpu.sync_copy(rowid_hbm, rows_v)
    pltpu.sync_copy(zeros_hbm, hist_v)
    pltpu.sync_copy(zeros_hbm.at[pl.ds(0, 5)], sdeg.at[pl.ds(sid * 5, 5)])
    plsc.subcore_barrier()
    ones = jnp.ones((16,), jnp.float32)

    def step(i, carry):
        idx = dst_v[pl.ds(i * 16, 16)]
        hi = lax.shift_right_logical(idx, 7)
        lo = lax.bitwise_and(idx, 127)
        plsc.addupdate_scatter(hist_v, [hi, lo], ones)
        return carry

    lax.fori_loop(0, EPT // 16, step, 0)
    pltpu.sync_copy(hist_v, sdeg.at[rows_v], add=True)
    plsc.subcore_barrier()
    pltpu.sync_copy(sdeg.at[pl.ds(sid * 5, 5)], out_hbm.at[cid, pl.ds(sid * 5, 5)])


def _deg_call(dst_flat, rowid, z80):
    fn = pl.kernel(
        _deg_body,
        out_type=jax.ShapeDtypeStruct((2, NCH, 128), jnp.float32),
        mesh=_sc_mesh(),
        scratch_types=[
            pltpu.VMEM((EPT,), jnp.int32),
            pltpu.VMEM((NCH, 128), jnp.float32),
            pltpu.VMEM((NCH,), jnp.int32),
            pltpu.VMEM_SHARED((NCH, 128), jnp.float32),
        ],
    )
    return fn(dst_flat, rowid, z80)


def _edge_body(hp_hbm, src_hbm, dst_hbm, zeros_hbm, out_hbm, src_v, dst_v, rows_v, acc, sem):
    cid = lax.axis_index("c")
    sid = lax.axis_index("s")
    wid = cid * 16 + sid
    pltpu.sync_copy(src_hbm.at[wid], src_v)
    pltpu.sync_copy(dst_hbm.at[wid], dst_v)
    pltpu.sync_copy(zeros_hbm, acc.at[pl.ds(sid * 640, 640)])
    plsc.subcore_barrier()

    def step(c, carry):
        pltpu.async_copy(hp_hbm.at[src_v.at[c]], rows_v, sem).wait()
        pltpu.sync_copy(rows_v, acc.at[dst_v.at[c]], add=True)
        return carry

    lax.fori_loop(0, NCH, step, 0)
    plsc.subcore_barrier()
    pltpu.sync_copy(acc.at[pl.ds(sid * 640, 640)], out_hbm.at[cid, pl.ds(sid * 640, 640)])


def _edge_call(d, hp, src3, dst3, zeros):
    fn = pl.kernel(
        _edge_body,
        out_type=jax.ShapeDtypeStruct((2, NACC, d), jnp.float32),
        mesh=_sc_mesh(),
        scratch_types=[
            pltpu.VMEM((NCH, CH), jnp.int32),
            pltpu.VMEM((NCH, CH), jnp.int32),
            pltpu.VMEM((CH, d), jnp.float32),
            pltpu.VMEM_SHARED((NACC, d), jnp.float32),
            pltpu.SemaphoreType.DMA,
        ],
    )
    return fn(hp, src3, dst3, zeros)


def _head_body(x_ref, d0_ref, d1_ref, w_ref, hp_ref, dinv_ref):
    deg = d0_ref[...] + d1_ref[...] + 1.0
    dinv = lax.rsqrt(deg)
    rows = lax.broadcasted_iota(jnp.int32, (NPAD, 1), 0)
    dinv = jnp.where(rows < N, dinv, 0.0)
    hp_ref[...] = (
        jnp.dot(x_ref[...], w_ref[...], preferred_element_type=jnp.float32) * dinv
    )
    dinv_ref[...] = dinv


def _mid_body(p_ref, hp_ref, dinv_ref, b_ref, w_ref, o_ref):
    h = dinv_ref[...] * (p_ref[0] + p_ref[1] + hp_ref[...]) + b_ref[...]
    h = jnp.maximum(h, 0.0)
    o_ref[...] = (
        jnp.dot(h, w_ref[...], preferred_element_type=jnp.float32) * dinv_ref[...]
    )


def _tail_body(p_ref, hp_ref, dinv_ref, b_ref, batch_ref, w1_ref, b1_ref, w2_ref, b2_ref, o_ref):
    h = dinv_ref[...] * (p_ref[0] + p_ref[1] + hp_ref[...]) + b_ref[...]
    h = jnp.maximum(h, 0.0)
    gids = lax.broadcasted_iota(jnp.int32, (64, NPAD), 0)
    seg = jnp.where(batch_ref[...] == gids, 1.0, 0.0)
    sums = jnp.dot(seg, h, preferred_element_type=jnp.float32)
    cnts = jnp.sum(seg, axis=1, keepdims=True)
    pooled = sums / jnp.maximum(cnts, 1.0)
    t = jnp.dot(pooled, w1_ref[...], preferred_element_type=jnp.float32) + b1_ref[...]
    t = jnp.maximum(t, 0.0)
    o_ref[...] = jnp.dot(t, w2_ref[...], preferred_element_type=jnp.float32) + b2_ref[...]


def kernel(x, edge_index, batch, W1, b1, W2, b2, W3, b3, fc1_W, fc1_b, fc2_W, fc2_b):
    f32, i32 = jnp.float32, jnp.int32
    src = edge_index[0]
    dst = edge_index[1]
    npad_e = EPAD - src.shape[0]
    padv = jnp.full((npad_e,), JUNK, i32)
    srcf = jnp.concatenate([src, padv])
    dstf = jnp.concatenate([dst, padv])
    src3 = srcf.reshape(NT, NCH, CH)
    dst3 = dstf.reshape(NT, NCH, CH)
    dst2 = dstf.reshape(NT, EPT)
    rowid = jnp.arange(NCH, dtype=i32)
    z80 = jnp.zeros((NCH, 128), f32)
    z640 = jnp.zeros((640, 64), f32)
    x_p = jnp.concatenate([x, jnp.zeros((NPAD - N, x.shape[1]), f32)])
    batch_p = jnp.concatenate([batch, jnp.full((NPAD - N,), 64, i32)]).reshape(1, NPAD)

    degs = _deg_call(dst2, rowid, z80).reshape(2, NACC)[:, :NPAD]
    d0 = degs[0].reshape(NPAD, 1)
    d1 = degs[1].reshape(NPAD, 1)

    hp1, dinv = pl.pallas_call(
        _head_body,
        out_shape=(
            jax.ShapeDtypeStruct((NPAD, 32), f32),
            jax.ShapeDtypeStruct((NPAD, 1), f32),
        ),
    )(x_p, d0, d1, W1)

    p1 = _edge_call(32, hp1, src3, dst3, z640[:, :32])[:, :NPAD]
    hp2 = pl.pallas_call(
        _mid_body, out_shape=jax.ShapeDtypeStruct((NPAD, 48), f32)
    )(p1, hp1, dinv, b1.reshape(1, 32), W2)

    p2 = _edge_call(48, hp2, src3, dst3, z640[:, :48])[:, :NPAD]
    hp3 = pl.pallas_call(
        _mid_body, out_shape=jax.ShapeDtypeStruct((NPAD, 64), f32)
    )(p2, hp2, dinv, b2.reshape(1, 48), W3)

    p3 = _edge_call(64, hp3, src3, dst3, z640)[:, :NPAD]
    out = pl.pallas_call(
        _tail_body, out_shape=jax.ShapeDtypeStruct((64, 1), f32)
    )(
        p3, hp3, dinv, b3.reshape(1, 64), batch_p,
        fc1_W, fc1_b.reshape(1, 32), fc2_W, fc2_b.reshape(1, 1),
    )
    return out


# speedup vs baseline: 15.5315x; 15.5315x over previous
import functools

import jax
import jax.numpy as jnp
from jax import lax
from jax.experimental import pallas as pl
from jax.experimental.pallas import tpu as pltpu
from jax.experimental.pallas import tpu_sc as plsc

N = 10000
NPAD = 10112
NACC = 10240
NT = 32
NCH = 80
CH = 128
EPT = NCH * CH
EPAD = NT * EPT
JUNK = N


def _sc_mesh():
    return plsc.VectorSubcoreMesh(
        core_axis_name="c", subcore_axis_name="s", num_cores=2, num_subcores=16
    )


def _deg_body(dst_hbm, zeros_hbm, out_hbm, dst_v, hist_v):
    cid = lax.axis_index("c")
    sid = lax.axis_index("s")
    wid = cid * 16 + sid
    pltpu.sync_copy(dst_hbm.at[pl.ds(wid * EPT, EPT)], dst_v)
    pltpu.sync_copy(zeros_hbm, hist_v)
    ones = jnp.ones((16,), jnp.float32)

    def step(i, carry):
        idx = dst_v[pl.ds(i * 16, 16)]
        plsc.addupdate_scatter(hist_v, [idx], ones)
        return carry

    lax.fori_loop(0, EPT // 16, step, 0)
    pltpu.sync_copy(hist_v, out_hbm.at[pl.ds(wid * NACC, NACC)])


def _deg_call(dst_flat, zdeg):
    fn = pl.kernel(
        _deg_body,
        out_type=jax.ShapeDtypeStruct((NT * NACC,), jnp.float32),
        mesh=_sc_mesh(),
        scratch_types=[
            pltpu.VMEM((EPT,), jnp.int32),
            pltpu.VMEM((NACC,), jnp.float32),
        ],
        compiler_params=pltpu.CompilerParams(needs_layout_passes=False),
    )
    return fn(dst_flat, zdeg)


def _edge_body(hp_hbm, src_hbm, dst_hbm, zeros_hbm, out_hbm, src_v, dst_v, rows_v, acc, sem):
    cid = lax.axis_index("c")
    sid = lax.axis_index("s")
    wid = cid * 16 + sid
    pltpu.sync_copy(src_hbm.at[wid], src_v)
    pltpu.sync_copy(dst_hbm.at[wid], dst_v)
    pltpu.sync_copy(zeros_hbm, acc.at[pl.ds(sid * 640, 640)])
    plsc.subcore_barrier()

    def step(c, carry):
        pltpu.async_copy(hp_hbm.at[src_v.at[c]], rows_v, sem).wait()
        pltpu.sync_copy(rows_v, acc.at[dst_v.at[c]], add=True)
        return carry

    lax.fori_loop(0, NCH, step, 0)
    plsc.subcore_barrier()
    pltpu.sync_copy(acc.at[pl.ds(sid * 640, 640)], out_hbm.at[cid, pl.ds(sid * 640, 640)])


def _edge_call(d, hp, src3, dst3, zeros):
    fn = pl.kernel(
        _edge_body,
        out_type=jax.ShapeDtypeStruct((2, NACC, d), jnp.float32),
        mesh=_sc_mesh(),
        scratch_types=[
            pltpu.VMEM((NCH, CH), jnp.int32),
            pltpu.VMEM((NCH, CH), jnp.int32),
            pltpu.VMEM((CH, d), jnp.float32),
            pltpu.VMEM_SHARED((NACC, d), jnp.float32),
            pltpu.SemaphoreType.DMA,
        ],
        compiler_params=pltpu.CompilerParams(use_tc_tiling_on_sc=False),
    )
    return fn(hp, src3, dst3, zeros)


def _head_body(x_ref, dall_ref, w_ref, hp_ref, dinv_ref):
    ones = jnp.ones((NT, 1), jnp.float32)
    deg = lax.dot_general(
        dall_ref[...], ones, (((0,), (0,)), ((), ())),
        preferred_element_type=jnp.float32,
    )
    deg = deg[:NPAD] + 1.0
    dinv = lax.rsqrt(deg)
    rows = lax.broadcasted_iota(jnp.int32, (NPAD, 1), 0)
    dinv = jnp.where(rows < N, dinv, 0.0)
    hp_ref[...] = (
        jnp.dot(x_ref[...], w_ref[...], preferred_element_type=jnp.float32) * dinv
    )
    dinv_ref[...] = dinv


def _mid_body(p_ref, hp_ref, dinv_ref, b_ref, w_ref, o_ref):
    h = dinv_ref[...] * (p_ref[0] + p_ref[1] + hp_ref[...]) + b_ref[...]
    h = jnp.maximum(h, 0.0)
    o_ref[...] = (
        jnp.dot(h, w_ref[...], preferred_element_type=jnp.float32) * dinv_ref[...]
    )


def _tail_body(p_ref, hp_ref, dinv_ref, b_ref, batch_ref, w1_ref, b1_ref, w2_ref, b2_ref, o_ref):
    h = dinv_ref[...] * (p_ref[0] + p_ref[1] + hp_ref[...]) + b_ref[...]
    h = jnp.maximum(h, 0.0)
    gids = lax.broadcasted_iota(jnp.int32, (64, NPAD), 0)
    seg = jnp.where(batch_ref[...] == gids, 1.0, 0.0)
    sums = jnp.dot(seg, h, preferred_element_type=jnp.float32)
    cnts = jnp.sum(seg, axis=1, keepdims=True)
    pooled = sums / jnp.maximum(cnts, 1.0)
    t = jnp.dot(pooled, w1_ref[...], preferred_element_type=jnp.float32) + b1_ref[...]
    t = jnp.maximum(t, 0.0)
    o_ref[...] = jnp.dot(t, w2_ref[...], preferred_element_type=jnp.float32) + b2_ref[...]


def kernel(x, edge_index, batch, W1, b1, W2, b2, W3, b3, fc1_W, fc1_b, fc2_W, fc2_b):
    f32, i32 = jnp.float32, jnp.int32
    src = edge_index[0]
    dst = edge_index[1]
    npad_e = EPAD - src.shape[0]
    padv = jnp.full((npad_e,), JUNK, i32)
    srcf = jnp.concatenate([src, padv])
    dstf = jnp.concatenate([dst, padv])
    src3 = srcf.reshape(NT, NCH, CH)
    dst3 = dstf.reshape(NT, NCH, CH)
    zdeg = jnp.zeros((NACC,), f32)
    z640 = jnp.zeros((640, 64), f32)
    x_p = jnp.concatenate([x, jnp.zeros((NPAD - N, x.shape[1]), f32)])
    batch_p = jnp.concatenate([batch, jnp.full((NPAD - N,), 64, i32)]).reshape(1, NPAD)

    d_all = _deg_call(dstf, zdeg).reshape(NT, NACC)

    hp1, dinv = pl.pallas_call(
        _head_body,
        out_shape=(
            jax.ShapeDtypeStruct((NPAD, 32), f32),
            jax.ShapeDtypeStruct((NPAD, 1), f32),
        ),
    )(x_p, d_all, W1)

    p1 = _edge_call(32, hp1, src3, dst3, z640[:, :32])[:, :NPAD]
    hp2 = pl.pallas_call(
        _mid_body, out_shape=jax.ShapeDtypeStruct((NPAD, 48), f32)
    )(p1, hp1, dinv, b1.reshape(1, 32), W2)

    p2 = _edge_call(48, hp2, src3, dst3, z640[:, :48])[:, :NPAD]
    hp3 = pl.pallas_call(
        _mid_body, out_shape=jax.ShapeDtypeStruct((NPAD, 64), f32)
    )(p2, hp2, dinv, b2.reshape(1, 48), W3)

    p3 = _edge_call(64, hp3, src3, dst3, z640)[:, :NPAD]
    out = pl.pallas_call(
        _tail_body, out_shape=jax.ShapeDtypeStruct((64, 1), f32)
    )(
        p3, hp3, dinv, b3.reshape(1, 64), batch_p,
        fc1_W, fc1_b.reshape(1, 32), fc2_W, fc2_b.reshape(1, 1),
    )
    return out


# speedup vs baseline: 18.6431x; 1.2003x over previous
import functools

import jax
import jax.numpy as jnp
from jax import lax
from jax.experimental import pallas as pl
from jax.experimental.pallas import tpu as pltpu
from jax.experimental.pallas import tpu_sc as plsc

N = 10000
NPAD = 10112
NACC = 10240
NT = 32
NCH = 80
CH = 128
EPT = NCH * CH
EPAD = NT * EPT
JUNK = N


def _sc_mesh():
    return plsc.VectorSubcoreMesh(
        core_axis_name="c", subcore_axis_name="s", num_cores=2, num_subcores=16
    )


def _deg_body(dst_hbm, zeros_hbm, out_hbm, dst_v, hist_v):
    cid = lax.axis_index("c")
    sid = lax.axis_index("s")
    wid = cid * 16 + sid
    pltpu.sync_copy(dst_hbm.at[pl.ds(wid * EPT, EPT)], dst_v)
    pltpu.sync_copy(zeros_hbm, hist_v)
    ones = jnp.ones((16,), jnp.float32)

    def step(i, carry):
        idx = dst_v[pl.ds(i * 16, 16)]
        plsc.addupdate_scatter(hist_v, [idx], ones)
        return carry

    lax.fori_loop(0, EPT // 16, step, 0)
    pltpu.sync_copy(hist_v, out_hbm.at[pl.ds(wid * NACC, NACC)])


def _deg_call(dst_flat, zdeg):
    fn = pl.kernel(
        _deg_body,
        out_type=jax.ShapeDtypeStruct((NT * NACC,), jnp.float32),
        mesh=_sc_mesh(),
        scratch_types=[
            pltpu.VMEM((EPT,), jnp.int32),
            pltpu.VMEM((NACC,), jnp.float32),
        ],
        compiler_params=pltpu.CompilerParams(needs_layout_passes=False),
    )
    return fn(dst_flat, zdeg)


def _edge_body(hp_hbm, src_hbm, dst_hbm, zeros_hbm, out_hbm,
               src_v, dst_v, rows_a, rows_b, acc, sem_a, sem_b):
    cid = lax.axis_index("c")
    sid = lax.axis_index("s")
    wid = cid * 16 + sid
    pltpu.sync_copy(src_hbm.at[wid], src_v)
    pltpu.sync_copy(dst_hbm.at[wid], dst_v)
    pltpu.sync_copy(zeros_hbm, acc.at[pl.ds(sid * 640, 640)])
    plsc.subcore_barrier()

    def gather(c, buf, sem):
        pltpu.async_copy(hp_hbm.at[src_v.at[c]], buf, sem)

    def wait(buf, sem):
        pltpu.make_async_copy(hp_hbm.at[src_v.at[0]], buf, sem).wait()

    gather(0, rows_a, sem_a)

    def step(i, carry):
        c0 = i * 2
        c1 = c0 + 1
        gather(c1, rows_b, sem_b)
        wait(rows_a, sem_a)
        pltpu.sync_copy(rows_a, acc.at[dst_v.at[c0]], add=True)

        @pl.when(c0 + 2 < NCH)
        def _pref():
            gather(c0 + 2, rows_a, sem_a)

        wait(rows_b, sem_b)
        pltpu.sync_copy(rows_b, acc.at[dst_v.at[c1]], add=True)
        return carry

    lax.fori_loop(0, NCH // 2, step, 0)
    plsc.subcore_barrier()
    pltpu.sync_copy(acc.at[pl.ds(sid * 640, 640)], out_hbm.at[cid, pl.ds(sid * 640, 640)])


def _edge_call(d, hp, src3, dst3, zeros):
    fn = pl.kernel(
        _edge_body,
        out_type=jax.ShapeDtypeStruct((2, NACC, d), jnp.float32),
        mesh=_sc_mesh(),
        scratch_types=[
            pltpu.VMEM((NCH, CH), jnp.int32),
            pltpu.VMEM((NCH, CH), jnp.int32),
            pltpu.VMEM((CH, d), jnp.float32),
            pltpu.VMEM((CH, d), jnp.float32),
            pltpu.VMEM_SHARED((NACC, d), jnp.float32),
            pltpu.SemaphoreType.DMA,
            pltpu.SemaphoreType.DMA,
        ],
        compiler_params=pltpu.CompilerParams(use_tc_tiling_on_sc=False),
    )
    return fn(hp, src3, dst3, zeros)


def _head_body(x_ref, dall_ref, w_ref, hp_ref, dinv_ref):
    ones = jnp.ones((NT, 1), jnp.float32)
    deg = lax.dot_general(
        dall_ref[...], ones, (((0,), (0,)), ((), ())),
        preferred_element_type=jnp.float32,
    )
    deg = deg[:NPAD] + 1.0
    dinv = lax.rsqrt(deg)
    rows = lax.broadcasted_iota(jnp.int32, (NPAD, 1), 0)
    dinv = jnp.where(rows < N, dinv, 0.0)
    hp_ref[...] = (
        jnp.dot(x_ref[...], w_ref[...], preferred_element_type=jnp.float32) * dinv
    )
    dinv_ref[...] = dinv


def _mid_body(p_ref, hp_ref, dinv_ref, b_ref, w_ref, o_ref):
    h = dinv_ref[...] * (p_ref[0] + p_ref[1] + hp_ref[...]) + b_ref[...]
    h = jnp.maximum(h, 0.0)
    o_ref[...] = (
        jnp.dot(h, w_ref[...], preferred_element_type=jnp.float32) * dinv_ref[...]
    )


def _tail_body(p_ref, hp_ref, dinv_ref, b_ref, batch_ref, w1_ref, b1_ref, w2_ref, b2_ref, o_ref):
    h = dinv_ref[...] * (p_ref[0] + p_ref[1] + hp_ref[...]) + b_ref[...]
    h = jnp.maximum(h, 0.0)
    gids = lax.broadcasted_iota(jnp.int32, (64, NPAD), 0)
    seg = jnp.where(batch_ref[...] == gids, 1.0, 0.0)
    sums = jnp.dot(seg, h, preferred_element_type=jnp.float32)
    cnts = jnp.sum(seg, axis=1, keepdims=True)
    pooled = sums / jnp.maximum(cnts, 1.0)
    t = jnp.dot(pooled, w1_ref[...], preferred_element_type=jnp.float32) + b1_ref[...]
    t = jnp.maximum(t, 0.0)
    o_ref[...] = jnp.dot(t, w2_ref[...], preferred_element_type=jnp.float32) + b2_ref[...]


def kernel(x, edge_index, batch, W1, b1, W2, b2, W3, b3, fc1_W, fc1_b, fc2_W, fc2_b):
    f32, i32 = jnp.float32, jnp.int32
    src = edge_index[0]
    dst = edge_index[1]
    npad_e = EPAD - src.shape[0]
    padv = jnp.full((npad_e,), JUNK, i32)
    srcf = jnp.concatenate([src, padv])
    dstf = jnp.concatenate([dst, padv])
    src3 = srcf.reshape(NT, NCH, CH)
    dst3 = dstf.reshape(NT, NCH, CH)
    zdeg = jnp.zeros((NACC,), f32)
    z640 = jnp.zeros((640, 64), f32)
    x_p = jnp.concatenate([x, jnp.zeros((NPAD - N, x.shape[1]), f32)])
    batch_p = jnp.concatenate([batch, jnp.full((NPAD - N,), 64, i32)]).reshape(1, NPAD)

    d_all = _deg_call(dstf, zdeg).reshape(NT, NACC)

    hp1, dinv = pl.pallas_call(
        _head_body,
        out_shape=(
            jax.ShapeDtypeStruct((NPAD, 32), f32),
            jax.ShapeDtypeStruct((NPAD, 1), f32),
        ),
    )(x_p, d_all, W1)

    p1 = _edge_call(32, hp1, src3, dst3, z640[:, :32])[:, :NPAD]
    hp2 = pl.pallas_call(
        _mid_body, out_shape=jax.ShapeDtypeStruct((NPAD, 48), f32)
    )(p1, hp1, dinv, b1.reshape(1, 32), W2)

    p2 = _edge_call(48, hp2, src3, dst3, z640[:, :48])[:, :NPAD]
    hp3 = pl.pallas_call(
        _mid_body, out_shape=jax.ShapeDtypeStruct((NPAD, 64), f32)
    )(p2, hp2, dinv, b2.reshape(1, 48), W3)

    p3 = _edge_call(64, hp3, src3, dst3, z640)[:, :NPAD]
    out = pl.pallas_call(
        _tail_body, out_shape=jax.ShapeDtypeStruct((64, 1), f32)
    )(
        p3, hp3, dinv, b3.reshape(1, 64), batch_p,
        fc1_W, fc1_b.reshape(1, 32), fc2_W, fc2_b.reshape(1, 1),
    )
    return out


# speedup vs baseline: 18.9139x; 1.0145x over previous
import functools

import jax
import jax.numpy as jnp
from jax import lax
from jax.experimental import pallas as pl
from jax.experimental.pallas import tpu as pltpu
from jax.experimental.pallas import tpu_sc as plsc

N = 10000
NPAD = 10112
NACC = 10240
NT = 32
NCH = 80
CH = 128
EPT = NCH * CH
EPAD = NT * EPT
JUNK = N


def _sc_mesh():
    return plsc.VectorSubcoreMesh(
        core_axis_name="c", subcore_axis_name="s", num_cores=2, num_subcores=16
    )


def _deg_body(dst_hbm, zeros_hbm, out_hbm, dst_v, hist_v):
    cid = lax.axis_index("c")
    sid = lax.axis_index("s")
    wid = cid * 16 + sid
    pltpu.sync_copy(dst_hbm.at[pl.ds(wid * EPT, EPT)], dst_v)
    pltpu.sync_copy(zeros_hbm, hist_v)
    ones = jnp.ones((16,), jnp.float32)

    def step(i, carry):
        idx = dst_v[pl.ds(i * 16, 16)]
        plsc.addupdate_scatter(hist_v, [idx], ones)
        return carry

    lax.fori_loop(0, EPT // 16, step, 0)
    pltpu.sync_copy(hist_v, out_hbm.at[pl.ds(wid * NACC, NACC)])


def _deg_call(dst_flat, zdeg):
    fn = pl.kernel(
        _deg_body,
        out_type=jax.ShapeDtypeStruct((NT * NACC,), jnp.float32),
        mesh=_sc_mesh(),
        scratch_types=[
            pltpu.VMEM((EPT,), jnp.int32),
            pltpu.VMEM((NACC,), jnp.float32),
        ],
        compiler_params=pltpu.CompilerParams(needs_layout_passes=False),
    )
    return fn(dst_flat, zdeg)


NB = 8
GAHEAD = 4


def _edge_body(hp_hbm, src_hbm, dst_hbm, zeros_hbm, out_hbm, src_v, dst_v, *rest):
    bufs = rest[0:NB]
    acc = rest[NB]
    gsems = rest[NB + 1:NB + 1 + NB]
    ssems = rest[NB + 1 + NB:NB + 1 + 2 * NB]
    cid = lax.axis_index("c")
    sid = lax.axis_index("s")
    wid = cid * 16 + sid
    pltpu.sync_copy(src_hbm.at[wid], src_v)
    pltpu.sync_copy(dst_hbm.at[wid], dst_v)
    pltpu.sync_copy(zeros_hbm, acc.at[pl.ds(sid * 640, 640)])
    plsc.subcore_barrier()

    def gather(c, buf, sem):
        pltpu.async_copy(hp_hbm.at[src_v.at[c]], buf, sem)

    def wait_g(buf, sem):
        pltpu.make_async_copy(hp_hbm.at[src_v.at[0]], buf, sem).wait()

    def scat(c, buf, sem):
        pltpu.async_copy(buf, acc.at[dst_v.at[c]], sem, add=True)

    def wait_s(buf, sem):
        pltpu.make_async_copy(buf, acc.at[dst_v.at[0]], sem).wait()

    for k in range(GAHEAD):
        gather(k, bufs[k], gsems[k])

    def round_(i, carry):
        for k in range(NB):
            c = i * NB + k
            nc = c + GAHEAD
            j = (k + GAHEAD) % NB

            @pl.when((nc < NCH) & (nc >= NB))
            def _drain(j=j):
                wait_s(bufs[j], ssems[j])

            @pl.when(nc < NCH)
            def _pref(j=j, nc=nc):
                gather(nc, bufs[j], gsems[j])

            wait_g(bufs[k], gsems[k])
            scat(c, bufs[k], ssems[k])
        return carry

    lax.fori_loop(0, NCH // NB, round_, 0)
    for k in range(NB):
        wait_s(bufs[k], ssems[k])
    plsc.subcore_barrier()
    pltpu.sync_copy(acc.at[pl.ds(sid * 640, 640)], out_hbm.at[cid, pl.ds(sid * 640, 640)])


def _edge_call(d, hp, src3, dst3, zeros):
    fn = pl.kernel(
        _edge_body,
        out_type=jax.ShapeDtypeStruct((2, NACC, d), jnp.float32),
        mesh=_sc_mesh(),
        scratch_types=[
            pltpu.VMEM((NCH, CH), jnp.int32),
            pltpu.VMEM((NCH, CH), jnp.int32),
            *[pltpu.VMEM((CH, d), jnp.float32) for _ in range(NB)],
            pltpu.VMEM_SHARED((NACC, d), jnp.float32),
            *[pltpu.SemaphoreType.DMA for _ in range(2 * NB)],
        ],
        compiler_params=pltpu.CompilerParams(use_tc_tiling_on_sc=False),
    )
    return fn(hp, src3, dst3, zeros)


def _head_body(x_ref, dall_ref, w_ref, hp_ref, dinv_ref):
    ones = jnp.ones((NT, 1), jnp.float32)
    deg = lax.dot_general(
        dall_ref[...], ones, (((0,), (0,)), ((), ())),
        preferred_element_type=jnp.float32,
    )
    deg = deg[:NPAD] + 1.0
    dinv = lax.rsqrt(deg)
    rows = lax.broadcasted_iota(jnp.int32, (NPAD, 1), 0)
    dinv = jnp.where(rows < N, dinv, 0.0)
    hp_ref[...] = (
        jnp.dot(x_ref[...], w_ref[...], preferred_element_type=jnp.float32) * dinv
    )
    dinv_ref[...] = dinv


def _mid_body(p_ref, hp_ref, dinv_ref, b_ref, w_ref, o_ref):
    h = dinv_ref[...] * (p_ref[0] + p_ref[1] + hp_ref[...]) + b_ref[...]
    h = jnp.maximum(h, 0.0)
    o_ref[...] = (
        jnp.dot(h, w_ref[...], preferred_element_type=jnp.float32) * dinv_ref[...]
    )


def _tail_body(p_ref, hp_ref, dinv_ref, b_ref, batch_ref, w1_ref, b1_ref, w2_ref, b2_ref, o_ref):
    h = dinv_ref[...] * (p_ref[0] + p_ref[1] + hp_ref[...]) + b_ref[...]
    h = jnp.maximum(h, 0.0)
    gids = lax.broadcasted_iota(jnp.int32, (64, NPAD), 0)
    seg = jnp.where(batch_ref[...] == gids, 1.0, 0.0)
    sums = jnp.dot(seg, h, preferred_element_type=jnp.float32)
    cnts = jnp.sum(seg, axis=1, keepdims=True)
    pooled = sums / jnp.maximum(cnts, 1.0)
    t = jnp.dot(pooled, w1_ref[...], preferred_element_type=jnp.float32) + b1_ref[...]
    t = jnp.maximum(t, 0.0)
    o_ref[...] = jnp.dot(t, w2_ref[...], preferred_element_type=jnp.float32) + b2_ref[...]


def kernel(x, edge_index, batch, W1, b1, W2, b2, W3, b3, fc1_W, fc1_b, fc2_W, fc2_b):
    f32, i32 = jnp.float32, jnp.int32
    src = edge_index[0]
    dst = edge_index[1]
    npad_e = EPAD - src.shape[0]
    padv = jnp.full((npad_e,), JUNK, i32)
    srcf = jnp.concatenate([src, padv])
    dstf = jnp.concatenate([dst, padv])
    src3 = srcf.reshape(NT, NCH, CH)
    dst3 = dstf.reshape(NT, NCH, CH)
    zdeg = jnp.zeros((NACC,), f32)
    z640 = jnp.zeros((640, 64), f32)
    x_p = jnp.concatenate([x, jnp.zeros((NPAD - N, x.shape[1]), f32)])
    batch_p = jnp.concatenate([batch, jnp.full((NPAD - N,), 64, i32)]).reshape(1, NPAD)

    d_all = _deg_call(dstf, zdeg).reshape(NT, NACC)

    hp1, dinv = pl.pallas_call(
        _head_body,
        out_shape=(
            jax.ShapeDtypeStruct((NPAD, 32), f32),
            jax.ShapeDtypeStruct((NPAD, 1), f32),
        ),
    )(x_p, d_all, W1)

    p1 = _edge_call(32, hp1, src3, dst3, z640[:, :32])[:, :NPAD]
    hp2 = pl.pallas_call(
        _mid_body, out_shape=jax.ShapeDtypeStruct((NPAD, 48), f32)
    )(p1, hp1, dinv, b1.reshape(1, 32), W2)

    p2 = _edge_call(48, hp2, src3, dst3, z640[:, :48])[:, :NPAD]
    hp3 = pl.pallas_call(
        _mid_body, out_shape=jax.ShapeDtypeStruct((NPAD, 64), f32)
    )(p2, hp2, dinv, b2.reshape(1, 48), W3)

    p3 = _edge_call(64, hp3, src3, dst3, z640)[:, :NPAD]
    out = pl.pallas_call(
        _tail_body, out_shape=jax.ShapeDtypeStruct((64, 1), f32)
    )(
        p3, hp3, dinv, b3.reshape(1, 64), batch_p,
        fc1_W, fc1_b.reshape(1, 32), fc2_W, fc2_b.reshape(1, 1),
    )
    return out
